# baseline (speedup 1.0000x reference)
import dataclasses
import functools

import jax
import jax.numpy as jnp
from jax import lax
from jax.experimental import pallas as pl
from jax.experimental.pallas import tpu as pltpu
from jax.experimental.pallas import tpu_sc as plsc

NUM_EXP = 8
LANES = 16


def _logits_body(x_ref, w_ref, b_ref, o_ref):
    blk = x_ref.shape[0]
    lgt = lax.dot_general(
        w_ref[...], x_ref[...], (((1,), (1,)), ((), ())),
        preferred_element_type=jnp.float32,
    ) + b_ref[...]
    for j in range(blk // 128):
        o_ref[j] = lgt[:, j * 128:(j + 1) * 128]


def _tc_logits(x2d, W, b2d, blk):
    T, D = x2d.shape
    E = W.shape[0]
    return pl.pallas_call(
        _logits_body,
        grid=(T // blk,),
        in_specs=[
            pl.BlockSpec((blk, D), lambda i: (i, 0)),
            pl.BlockSpec((E, D), lambda i: (0, 0)),
            pl.BlockSpec((E, 1), lambda i: (0, 0)),
        ],
        out_specs=pl.BlockSpec((blk // 128, E, 128), lambda i: (i, 0, 0)),
        out_shape=jax.ShapeDtypeStruct((T // 128, E, 128), jnp.float32),
    )(x2d, W, b2d)


def _sc_select(lg3):
    C, E, L128 = lg3.shape
    T = C * L128
    mesh = plsc.VectorSubcoreMesh(core_axis_name="c", subcore_axis_name="s")
    nw = mesh.num_cores * mesh.num_subcores
    cpw = C // nw
    tpw = T // nw

    cp = pltpu.CompilerParams()
    if "needs_layout_passes" in pltpu.CompilerParams.__dataclass_fields__:
        cp = dataclasses.replace(cp, needs_layout_passes=False)
    if "use_tc_tiling_on_sc" in pltpu.CompilerParams.__dataclass_fields__:
        cp = dataclasses.replace(cp, use_tc_tiling_on_sc=False)

    @functools.partial(
        pl.kernel,
        compiler_params=cp,
        out_type=(
            jax.ShapeDtypeStruct((2 * T,), jnp.float32),
            jax.ShapeDtypeStruct((2 * T,), jnp.int32),
        ),
        mesh=mesh,
        scratch_types=[
            pltpu.VMEM((cpw, E, 128), jnp.float32),
            pltpu.VMEM((2 * tpw,), jnp.float32),
            pltpu.VMEM((2 * tpw,), jnp.int32),
        ],
    )
    def k(lg_hbm, w_hbm, e_hbm, lg_v, w_v, e_v):
        wid = lax.axis_index("s") * mesh.num_cores + lax.axis_index("c")
        pltpu.sync_copy(lg_hbm.at[pl.ds(wid * cpw, cpw)], lg_v)

        @pl.loop(0, cpw)
        def _(c):
            @pl.loop(0, 128, step=LANES)
            def _(l0):
                l = [lg_v[c, e, pl.ds(l0, LANES)] for e in range(E)]
                m1 = l[0]
                for e in range(1, E):
                    m1 = jnp.maximum(m1, l[e])
                i1 = jnp.zeros((LANES,), jnp.int32)
                for e in range(E - 1, -1, -1):
                    i1 = jnp.where(l[e] == m1, jnp.int32(e), i1)
                neg = jnp.float32(-jnp.inf)
                m2 = jnp.where(i1 == 0, neg, l[0])
                for e in range(1, E):
                    m2 = jnp.maximum(m2, jnp.where(i1 == e, neg, l[e]))
                i2 = jnp.zeros((LANES,), jnp.int32)
                for e in range(E - 1, -1, -1):
                    i2 = jnp.where((l[e] == m2) & (i1 != e), jnp.int32(e), i2)
                r = jnp.exp(m2 - m1)
                w1 = 1.0 / (1.0 + r)
                w2 = r / (1.0 + r)
                flat = 2 * (c * 128 + l0) + 2 * lax.iota(jnp.int32, LANES)
                plsc.store_scatter(w_v, [flat], w1)
                plsc.store_scatter(w_v, [flat + 1], w2)
                plsc.store_scatter(e_v, [flat], i1)
                plsc.store_scatter(e_v, [flat + 1], i2)

        pltpu.sync_copy(w_v, w_hbm.at[pl.ds(wid * 2 * tpw, 2 * tpw)])
        pltpu.sync_copy(e_v, e_hbm.at[pl.ds(wid * 2 * tpw, 2 * tpw)])

    return k(lg3)


def kernel(hidden_states, W, b):
    B, S, D = hidden_states.shape
    x2d = hidden_states.reshape(B * S, D)
    lg3 = _tc_logits(x2d, W, b.reshape(NUM_EXP, 1), blk=4096)
    wf, ef = _sc_select(lg3)
    return wf.reshape(B * S, 2), ef.reshape(B * S, 2)


# speedup vs baseline: 2.0184x; 2.0184x over previous
import dataclasses
import functools

import jax
import jax.numpy as jnp
from jax import lax
from jax.experimental import pallas as pl
from jax.experimental.pallas import tpu as pltpu
from jax.experimental.pallas import tpu_sc as plsc

NUM_EXP = 8
LANES = 16


def _logits_body(x_ref, w_ref, b_ref, o_ref):
    blk = x_ref.shape[0]
    lgt = lax.dot_general(
        w_ref[...], x_ref[...], (((1,), (1,)), ((), ())),
        preferred_element_type=jnp.float32,
    ) + b_ref[...]
    for j in range(blk // 128):
        o_ref[j] = lgt[:, j * 128:(j + 1) * 128]


def _tc_logits(x2d, W, b2d, blk):
    T, D = x2d.shape
    E = W.shape[0]
    return pl.pallas_call(
        _logits_body,
        grid=(T // blk,),
        in_specs=[
            pl.BlockSpec((blk, D), lambda i: (i, 0)),
            pl.BlockSpec((E, D), lambda i: (0, 0)),
            pl.BlockSpec((E, 1), lambda i: (0, 0)),
        ],
        out_specs=pl.BlockSpec((blk // 128, E, 128), lambda i: (i, 0, 0)),
        out_shape=jax.ShapeDtypeStruct((T // 128, E, 128), jnp.float32),
    )(x2d, W, b2d)


def _sc_select(lg3):
    C, E, L128 = lg3.shape
    T = C * L128
    mesh = plsc.VectorSubcoreMesh(core_axis_name="c", subcore_axis_name="s")
    nw = mesh.num_cores * mesh.num_subcores
    cpw = C // nw
    tpw = T // nw

    cp = pltpu.CompilerParams()
    if "needs_layout_passes" in pltpu.CompilerParams.__dataclass_fields__:
        cp = dataclasses.replace(cp, needs_layout_passes=False)
    if "use_tc_tiling_on_sc" in pltpu.CompilerParams.__dataclass_fields__:
        cp = dataclasses.replace(cp, use_tc_tiling_on_sc=False)

    @functools.partial(
        pl.kernel,
        compiler_params=cp,
        out_type=(
            jax.ShapeDtypeStruct((C, 2, 128), jnp.float32),
            jax.ShapeDtypeStruct((C, 2, 128), jnp.int32),
        ),
        mesh=mesh,
        scratch_types=[
            pltpu.VMEM((cpw, E, 128), jnp.float32),
            pltpu.VMEM((cpw, 2, 128), jnp.float32),
            pltpu.VMEM((cpw, 2, 128), jnp.int32),
        ],
    )
    def k(lg_hbm, w_hbm, e_hbm, lg_v, w_v, e_v):
        wid = lax.axis_index("s") * mesh.num_cores + lax.axis_index("c")
        pltpu.sync_copy(lg_hbm.at[pl.ds(wid * cpw, cpw)], lg_v)

        @pl.loop(0, cpw)
        def _(c):
            @pl.loop(0, 128, step=LANES)
            def _(l0):
                l = [lg_v[c, e, pl.ds(l0, LANES)] for e in range(E)]
                m1 = l[0]
                for e in range(1, E):
                    m1 = jnp.maximum(m1, l[e])
                i1 = jnp.zeros((LANES,), jnp.int32)
                for e in range(E - 1, -1, -1):
                    i1 = jnp.where(l[e] == m1, jnp.int32(e), i1)
                neg = jnp.float32(-jnp.inf)
                m2 = jnp.where(i1 == 0, neg, l[0])
                for e in range(1, E):
                    m2 = jnp.maximum(m2, jnp.where(i1 == e, neg, l[e]))
                i2 = jnp.zeros((LANES,), jnp.int32)
                for e in range(E - 1, -1, -1):
                    i2 = jnp.where((l[e] == m2) & (i1 != e), jnp.int32(e), i2)
                r = jnp.exp(m2 - m1)
                w1 = 1.0 / (1.0 + r)
                w2 = r / (1.0 + r)
                w_v[c, 0, pl.ds(l0, LANES)] = w1
                w_v[c, 1, pl.ds(l0, LANES)] = w2
                e_v[c, 0, pl.ds(l0, LANES)] = i1
                e_v[c, 1, pl.ds(l0, LANES)] = i2

        pltpu.sync_copy(w_v, w_hbm.at[pl.ds(wid * cpw, cpw)])
        pltpu.sync_copy(e_v, e_hbm.at[pl.ds(wid * cpw, cpw)])

    return k(lg3)


def kernel(hidden_states, W, b):
    B, S, D = hidden_states.shape
    x2d = hidden_states.reshape(B * S, D)
    lg3 = _tc_logits(x2d, W, b.reshape(NUM_EXP, 1), blk=4096)
    wf, ef = _sc_select(lg3)
    wo = wf.transpose(0, 2, 1).reshape(B * S, 2)
    eo = ef.transpose(0, 2, 1).reshape(B * S, 2)
    return wo, eo


# speedup vs baseline: 2.0974x; 1.0391x over previous
import dataclasses
import functools

import jax
import jax.numpy as jnp
from jax import lax
from jax.experimental import pallas as pl
from jax.experimental.pallas import tpu as pltpu
from jax.experimental.pallas import tpu_sc as plsc

NUM_EXP = 8
LANES = 16


def _logits_body(x_ref, w_ref, b_ref, o_ref):
    blk = x_ref.shape[0]
    lgt = lax.dot_general(
        w_ref[...], x_ref[...], (((1,), (1,)), ((), ())),
        preferred_element_type=jnp.float32,
    ) + b_ref[...]
    for j in range(blk // 128):
        o_ref[j] = lgt[:, j * 128:(j + 1) * 128]


def _tc_logits(x2d, W, b2d, blk):
    T, D = x2d.shape
    E = W.shape[0]
    return pl.pallas_call(
        _logits_body,
        grid=(T // blk,),
        in_specs=[
            pl.BlockSpec((blk, D), lambda i: (i, 0)),
            pl.BlockSpec((E, D), lambda i: (0, 0)),
            pl.BlockSpec((E, 1), lambda i: (0, 0)),
        ],
        out_specs=pl.BlockSpec((blk // 128, E, 128), lambda i: (i, 0, 0)),
        out_shape=jax.ShapeDtypeStruct((T // 128, E, 128), jnp.float32),
    )(x2d, W, b2d)


def _sc_select(lg3):
    C, E, L128 = lg3.shape
    T = C * L128
    mesh = plsc.VectorSubcoreMesh(core_axis_name="c", subcore_axis_name="s")
    nw = mesh.num_cores * mesh.num_subcores
    cpw = C // nw
    tpw = T // nw

    cp = pltpu.CompilerParams()
    if "needs_layout_passes" in pltpu.CompilerParams.__dataclass_fields__:
        cp = dataclasses.replace(cp, needs_layout_passes=False)
    if "use_tc_tiling_on_sc" in pltpu.CompilerParams.__dataclass_fields__:
        cp = dataclasses.replace(cp, use_tc_tiling_on_sc=False)

    @functools.partial(
        pl.kernel,
        compiler_params=cp,
        out_type=(
            jax.ShapeDtypeStruct((C, 2, 128), jnp.float32),
            jax.ShapeDtypeStruct((C, 2, 128), jnp.int32),
        ),
        mesh=mesh,
        scratch_types=[
            pltpu.VMEM((cpw, E, 128), jnp.float32),
            pltpu.VMEM((cpw, 2, 128), jnp.float32),
            pltpu.VMEM((cpw, 2, 128), jnp.int32),
        ],
    )
    def k(lg_hbm, w_hbm, e_hbm, lg_v, w_v, e_v):
        wid = lax.axis_index("s") * mesh.num_cores + lax.axis_index("c")
        pltpu.sync_copy(lg_hbm.at[pl.ds(wid * cpw, cpw)], lg_v)

        @pl.loop(0, cpw)
        def _(c):
            @pl.loop(0, 128, step=LANES)
            def _(l0):
                l = [lg_v[c, e, pl.ds(l0, LANES)] for e in range(E)]
                m1 = l[0]
                for e in range(1, E):
                    m1 = jnp.maximum(m1, l[e])
                i1 = jnp.zeros((LANES,), jnp.int32)
                for e in range(E - 1, -1, -1):
                    i1 = jnp.where(l[e] == m1, jnp.int32(e), i1)
                neg = jnp.float32(-jnp.inf)
                m2 = jnp.where(i1 == 0, neg, l[0])
                for e in range(1, E):
                    m2 = jnp.maximum(m2, jnp.where(i1 == e, neg, l[e]))
                i2 = jnp.zeros((LANES,), jnp.int32)
                for e in range(E - 1, -1, -1):
                    i2 = jnp.where((l[e] == m2) & (i1 != e), jnp.int32(e), i2)
                r = jnp.exp(m2 - m1)
                w1 = 1.0 / (1.0 + r)
                w2 = r / (1.0 + r)
                w_v[c, 0, pl.ds(l0, LANES)] = w1
                w_v[c, 1, pl.ds(l0, LANES)] = w2
                e_v[c, 0, pl.ds(l0, LANES)] = i1
                e_v[c, 1, pl.ds(l0, LANES)] = i2

        pltpu.sync_copy(w_v, w_hbm.at[pl.ds(wid * cpw, cpw)])
        pltpu.sync_copy(e_v, e_hbm.at[pl.ds(wid * cpw, cpw)])

    return k(lg3)


def kernel(hidden_states, W, b):
    B, S, D = hidden_states.shape
    x2d = hidden_states.reshape(B * S, D)
    lg3 = _tc_logits(x2d, W, b.reshape(NUM_EXP, 1), blk=2048)
    wf, ef = _sc_select(lg3)
    wo = wf.transpose(0, 2, 1).reshape(B * S, 2)
    eo = ef.transpose(0, 2, 1).reshape(B * S, 2)
    return wo, eo
